# baseline (speedup 1.0000x reference)
import functools

import jax
import jax.numpy as jnp
from jax import lax
from jax.experimental import pallas as pl
from jax.experimental.pallas import tpu as pltpu
from jax.experimental.pallas import tpu_sc as plsc

_EPS = 1e-12
_NC = 2
_NS = 16
_CH = 32
_L = 16


def _sc_gather_pack(weight, ids):
    (n,) = ids.shape
    _, d = weight.shape
    nw = _NC * _NS
    per_w = n // nw
    assert n % (8 * nw) == 0 and per_w % _CH == 0

    mesh = plsc.VectorSubcoreMesh(core_axis_name="c", subcore_axis_name="s")
    nch = per_w // _CH

    @functools.partial(
        pl.kernel,
        out_type=jax.ShapeDtypeStruct((n // 2, d), jnp.int32),
        mesh=mesh,
        scratch_types=[
            pltpu.VMEM((per_w,), jnp.int32),
            pltpu.VMEM((_CH, d), jnp.int32),
            pltpu.VMEM((_CH, d), jnp.int32),
            pltpu.VMEM((_CH // 2, d), jnp.int32),
            pltpu.VMEM((_CH // 2, d), jnp.int32),
            pltpu.SemaphoreType.DMA,
            pltpu.SemaphoreType.DMA,
            pltpu.SemaphoreType.DMA,
            pltpu.SemaphoreType.DMA,
        ],
    )
    def k(table_hbm, idx_hbm, out_hbm, idx_v, rows_a, rows_b, pk_a, pk_b,
          gs_a, gs_b, ws_a, ws_b):
        wid = lax.axis_index("s") * _NC + lax.axis_index("c")
        base = wid * per_w
        pltpu.sync_copy(idx_hbm.at[pl.ds(base, per_w)], idx_v)

        rows_bufs = [(rows_a, gs_a), (rows_b, gs_b)]
        pk_bufs = [(pk_a, ws_a), (pk_b, ws_b)]
        gathers = [None] * nch
        writes = [None] * nch
        gathers[0] = pltpu.async_copy(
            table_hbm.at[idx_v.at[pl.ds(0, _CH)]], rows_a, gs_a
        )
        half = jnp.int32(0x8000)
        himask = jnp.int32(-0x10000)
        for j in range(nch):
            rows, _ = rows_bufs[j % 2]
            pk, ws = pk_bufs[j % 2]
            if j + 1 < nch:
                rows_n, gs_n = rows_bufs[(j + 1) % 2]
                gathers[j + 1] = pltpu.async_copy(
                    table_hbm.at[idx_v.at[pl.ds((j + 1) * _CH, _CH)]],
                    rows_n,
                    gs_n,
                )
            gathers[j].wait()
            if j >= 2:
                writes[j - 2].wait()

            @pl.loop(0, _CH // 2)
            def _(p):
                for g in range(d // _L):
                    c = g * _L
                    ua = rows[2 * p, pl.ds(c, _L)]
                    ub = rows[2 * p + 1, pl.ds(c, _L)]
                    lo = lax.shift_right_logical(ua + half, 16)
                    hi = (ub + half) & himask
                    pk[p, pl.ds(c, _L)] = lo | hi

            off = pl.multiple_of(base // 2 + j * (_CH // 2), _CH // 2)
            writes[j] = pltpu.async_copy(
                pk,
                out_hbm.at[pl.ds(off, _CH // 2)],
                ws,
            )
        if nch >= 2:
            writes[nch - 2].wait()
        writes[nch - 1].wait()

    return k(weight, ids)


def _ln_pair_body(pk_ref, p_ref, gamma_ref, beta_ref, o_ref):
    u = lax.bitcast_convert_type(pk_ref[...], jnp.uint32)
    x_even = lax.bitcast_convert_type(u << 16, jnp.float32)
    x_odd = lax.bitcast_convert_type(u & jnp.uint32(0xFFFF0000), jnp.float32)
    gamma = gamma_ref[...]
    beta = beta_ref[...]

    def ln(x):
        mean = jnp.mean(x, axis=-1, keepdims=True)
        xc = x - mean
        var = jnp.mean(xc * xc, axis=-1, keepdims=True)
        return (xc * lax.rsqrt(var + _EPS)) * gamma + beta

    o_ref[:, 0, :] = ln(x_even + p_ref[:, 0, :])
    o_ref[:, 1, :] = ln(x_odd + p_ref[:, 1, :])


def _tc_unpack_add_ln(packed, pos, gamma, beta, b, s, bsp=512):
    np_, d = packed.shape
    ppb = s // 2 // bsp
    pos_pairs = pos.reshape(s // 2, 2, d)
    out = pl.pallas_call(
        _ln_pair_body,
        grid=(ppb, b),
        in_specs=[
            pl.BlockSpec((bsp, d), lambda i, j: (j * ppb + i, 0)),
            pl.BlockSpec((bsp, 2, d), lambda i, j: (i, 0, 0)),
            pl.BlockSpec((d,), lambda i, j: (0,)),
            pl.BlockSpec((d,), lambda i, j: (0,)),
        ],
        out_specs=pl.BlockSpec((bsp, 2, d), lambda i, j: (j * ppb + i, 0, 0)),
        out_shape=jax.ShapeDtypeStruct((np_, 2, d), jnp.float32),
    )(packed, pos_pairs, gamma, beta)
    return out.reshape(b, s, d)


def kernel(input_ids, weight, position_embeddings, ln_gamma, ln_beta):
    b, s = input_ids.shape
    _, d = weight.shape
    ids = input_ids.reshape(-1).astype(jnp.int32)
    weight_i = lax.bitcast_convert_type(weight, jnp.int32)
    packed = _sc_gather_pack(weight_i, ids)
    return _tc_unpack_add_ln(
        packed, position_embeddings[:s], ln_gamma, ln_beta, b, s
    )


# speedup vs baseline: 5.6103x; 5.6103x over previous
import functools

import jax
import jax.numpy as jnp
from jax import lax
from jax.experimental import pallas as pl
from jax.experimental.pallas import tpu as pltpu
from jax.experimental.pallas import tpu_sc as plsc

_EPS = 1e-12
_NC = 2
_NS = 16
_CH = 32


def _sc_gather(weight, ids):
    (n,) = ids.shape
    _, d = weight.shape
    nw = _NC * _NS
    per_w = n // nw
    assert n % (8 * nw) == 0 and per_w % _CH == 0

    mesh = plsc.VectorSubcoreMesh(core_axis_name="c", subcore_axis_name="s")
    nch = per_w // _CH

    @functools.partial(
        pl.kernel,
        out_type=jax.ShapeDtypeStruct((n, d), jnp.float32),
        mesh=mesh,
        scratch_types=[
            pltpu.VMEM((per_w,), jnp.int32),
            pltpu.VMEM((_CH, d), jnp.float32),
            pltpu.VMEM((_CH, d), jnp.float32),
            pltpu.SemaphoreType.DMA,
            pltpu.SemaphoreType.DMA,
            pltpu.SemaphoreType.DMA,
            pltpu.SemaphoreType.DMA,
        ],
    )
    def k(table_hbm, idx_hbm, out_hbm, idx_v, rows_a, rows_b, gs_a, gs_b, ws_a, ws_b):
        wid = lax.axis_index("s") * _NC + lax.axis_index("c")
        base = wid * per_w
        pltpu.sync_copy(idx_hbm.at[pl.ds(base, per_w)], idx_v)

        bufs = [(rows_a, gs_a, ws_a), (rows_b, gs_b, ws_b)]
        gathers = [None] * nch
        writes = [None] * nch
        gathers[0] = pltpu.async_copy(
            table_hbm.at[idx_v.at[pl.ds(0, _CH)]], rows_a, gs_a
        )
        for j in range(nch):
            rows, _, ws = bufs[j % 2]
            if j + 1 < nch:
                rows_n, gs_n, _ = bufs[(j + 1) % 2]
                if j >= 1:
                    writes[j - 1].wait()
                gathers[j + 1] = pltpu.async_copy(
                    table_hbm.at[idx_v.at[pl.ds((j + 1) * _CH, _CH)]],
                    rows_n,
                    gs_n,
                )
            gathers[j].wait()
            writes[j] = pltpu.async_copy(
                rows, out_hbm.at[pl.ds(base + j * _CH, _CH)], ws
            )
        if nch >= 2:
            writes[nch - 2].wait()
        writes[nch - 1].wait()

    return k(weight, ids)


def _ln_body(g_ref, p_ref, gamma_ref, beta_ref, o_ref):
    x = g_ref[...] + p_ref[...]
    mean = jnp.mean(x, axis=-1, keepdims=True)
    xc = x - mean
    var = jnp.mean(xc * xc, axis=-1, keepdims=True)
    normed = xc * lax.rsqrt(var + _EPS)
    o_ref[...] = normed * gamma_ref[...] + beta_ref[...]


def _ln_body2(g_ref, p_ref, gamma_ref, beta_ref, o_ref):
    x = g_ref[...] + p_ref[...]
    mean = jnp.mean(x, axis=-1, keepdims=True)
    xc = x - mean
    var = jnp.mean(xc * xc, axis=-1, keepdims=True)
    o_ref[...] = (xc * lax.rsqrt(var + _EPS)) * gamma_ref[...] + beta_ref[...]


def _tc_add_ln(g3, pos, gamma, beta, bs=256):
    b, s, d = g3.shape
    return pl.pallas_call(
        _ln_body,
        grid=(s // bs, b),
        in_specs=[
            pl.BlockSpec((1, bs, d), lambda i, j: (j, i, 0)),
            pl.BlockSpec((bs, d), lambda i, j: (i, 0)),
            pl.BlockSpec((d,), lambda i, j: (0,)),
            pl.BlockSpec((d,), lambda i, j: (0,)),
        ],
        out_specs=pl.BlockSpec((1, bs, d), lambda i, j: (j, i, 0)),
        out_shape=jax.ShapeDtypeStruct((b, s, d), jnp.float32),
    )(g3, pos, gamma, beta)


def _tc_add_ln_flat(g2, pos, gamma, beta, b, s, bs=2048):
    _, d = g2.shape
    spb = s // bs
    out = pl.pallas_call(
        _ln_body2,
        grid=(spb, b),
        in_specs=[
            pl.BlockSpec((bs, d), lambda i, j: (j * spb + i, 0)),
            pl.BlockSpec((bs, d), lambda i, j: (i, 0)),
            pl.BlockSpec((d,), lambda i, j: (0,)),
            pl.BlockSpec((d,), lambda i, j: (0,)),
        ],
        out_specs=pl.BlockSpec((bs, d), lambda i, j: (j * spb + i, 0)),
        out_shape=jax.ShapeDtypeStruct((b * s, d), jnp.float32),
    )(g2, pos, gamma, beta)
    return out.reshape(b, s, d)


def kernel(input_ids, weight, position_embeddings, ln_gamma, ln_beta):
    b, s = input_ids.shape
    _, d = weight.shape
    ids = input_ids.reshape(-1).astype(jnp.int32)
    gathered = _sc_gather(weight, ids)
    return _tc_add_ln_flat(gathered, position_embeddings[:s], ln_gamma, ln_beta, b, s)
